# Initial kernel scaffold; baseline (speedup 1.0000x reference)
import jax
import jax.numpy as jnp
from jax.experimental import pallas as pl


def kernel(group_user_flat, user_seg_ids, behavior_items, behavior_counts, behavior_user_ids, user_table, item_table, lin_W, lin_b):
    raise NotImplementedError("write your pallas kernel here")


# speedup vs baseline: 2.4946x; 2.4946x over previous
import jax
import jax.numpy as jnp
from jax import lax
from jax.experimental import pallas as pl
from jax.experimental.pallas import tpu as pltpu
from jax.experimental.pallas import tpu_sc as plsc

N_GROUPS = 4096
TOTAL_USERS = 204800
TOTAL_BEHAVIORS = 2048000
EMB = 32

NC = 2
NS = 16
NW = NC * NS
UR = 3200
NRANGE = TOTAL_USERS // UR
ROUNDS = NRANGE // NW
T = 64
NCHUNK = UR // T
GSLICE = N_GROUPS // NS

_i32 = jnp.int32
_f32 = jnp.float32


def _sread(ref, idx):
    return ref[pl.ds(idx, 16)][0]


def _sc_body(starts_h, gu_h, seg_h, bi_h, bc_h, bu_h, utab_h, itab_h,
             outp_h, outc_h,
             uacc_s, gacc_s, gcnt_s,
             starts_v, bi_v, bc_v, bu_v, uofs_v, rows_v,
             gu_v, seg_v, urows_v, acc_v, fm_v, zc_v, zc16_v, ones_v, sem):
    c = lax.axis_index("c")
    s = lax.axis_index("s")
    wid = c * NS + s
    sbase = s * UR

    zero16 = jnp.zeros((16,), _f32)
    one16 = jnp.ones((16,), _f32)

    @pl.loop(0, T)
    def _fill(i):
        zc_v[i, pl.ds(0, 16)] = zero16
        zc_v[i, pl.ds(16, 16)] = zero16
        zc16_v[i, :] = zero16
        ones_v[i, :] = one16

    for t in range(GSLICE // T):
        pltpu.sync_copy(zc_v, gacc_s.at[pl.ds(s * GSLICE + t * T, T)])
        pltpu.sync_copy(zc16_v, gcnt_s.at[pl.ds(s * GSLICE + t * T, T)])
    pltpu.sync_copy(starts_h, starts_v)
    plsc.subcore_barrier()

    for j in range(ROUNDS):
        r = wid * ROUNDS + j
        lo = r * UR

        for t in range(NCHUNK):
            pltpu.sync_copy(zc_v, uacc_s.at[pl.ds(sbase + t * T, T)])

        sA = _sread(starts_v, r)
        eA = _sread(starts_v, r + 1)
        k0 = sA >> 6
        k1 = (eA + (T - 1)) >> 6

        @pl.loop(k0, k1)
        def _tile(k):
            base = k * T
            pltpu.sync_copy(bi_h.at[pl.ds(base, T)], bi_v)
            pltpu.sync_copy(bc_h.at[pl.ds(base, T)], bc_v)
            pltpu.sync_copy(bu_h.at[pl.ds(base, T)], bu_v)
            pltpu.async_copy(itab_h.at[bi_v], rows_v, sem).wait()
            for v8 in range(T // 16):
                sl = pl.ds(v8 * 16, 16)
                u = bu_v[sl] - lo
                valid = (u >= 0) & (u < UR)
                uofs_v[sl] = jnp.where(valid, u, 0) + sbase
                bc_v[sl] = jnp.where(valid, bc_v[sl], 0.0)

            @pl.loop(0, T)
            def _scale(rr):
                cidx = jnp.broadcast_to(rr, (16,))
                cv = plsc.load_gather(bc_v, [cidx])
                rows_v[rr, pl.ds(0, 16)] = rows_v[rr, pl.ds(0, 16)] * cv
                rows_v[rr, pl.ds(16, 16)] = rows_v[rr, pl.ds(16, 16)] * cv

            pltpu.sync_copy(rows_v, uacc_s.at[uofs_v], add=True)

        @pl.loop(0, NCHUNK)
        def _chunk(q):
            ub = lo + q * T
            pltpu.sync_copy(gu_h.at[pl.ds(ub, T)], gu_v)
            pltpu.sync_copy(seg_h.at[pl.ds(ub, T)], seg_v)
            pltpu.async_copy(utab_h.at[gu_v], urows_v, sem).wait()
            pltpu.sync_copy(uacc_s.at[pl.ds(sbase + q * T, T)], acc_v)
            for v8 in range(T // 16):
                sl = pl.ds(v8 * 16, 16)
                fm_v[sl] = jnp.where(gu_v[sl] != 0, 1.0, 0.0)

            @pl.loop(0, T)
            def _mul(rr):
                fidx = jnp.broadcast_to(rr, (16,))
                fv = plsc.load_gather(fm_v, [fidx])
                for h in (0, 16):
                    urows_v[rr, pl.ds(h, 16)] = (
                        urows_v[rr, pl.ds(h, 16)]
                        * acc_v[rr, pl.ds(h, 16)] * fv)

            pltpu.sync_copy(urows_v, gacc_s.at[seg_v], add=True)
            pltpu.sync_copy(ones_v, gcnt_s.at[seg_v], add=True)

    plsc.subcore_barrier()
    for t in range(GSLICE // T):
        off = s * GSLICE + t * T
        pltpu.sync_copy(gacc_s.at[pl.ds(off, T)], outp_h.at[c].at[pl.ds(off, T)])
        pltpu.sync_copy(gcnt_s.at[pl.ds(off, T)], outc_h.at[c].at[pl.ds(off, T)])


def _combine_body(p_ref, c_ref, o_ref):
    ps = p_ref[0] + p_ref[1]
    cnt = c_ref[0, :, 0:1] + c_ref[1, :, 0:1]
    o_ref[...] = ps / jnp.maximum(cnt, 1.0)


@jax.jit
def kernel(group_user_flat, user_seg_ids, behavior_items, behavior_counts,
           behavior_user_ids, user_table, item_table, lin_W, lin_b):
    del lin_W, lin_b
    bounds = jnp.arange(0, TOTAL_USERS + 1, UR, dtype=_i32)
    starts = jnp.searchsorted(behavior_user_ids, bounds).astype(_i32)
    starts = jnp.concatenate(
        [starts, jnp.full((80 - NRANGE - 1,), TOTAL_BEHAVIORS, _i32)])

    mesh = plsc.VectorSubcoreMesh(core_axis_name="c", subcore_axis_name="s")
    sc = pl.kernel(
        _sc_body,
        out_type=(
            jax.ShapeDtypeStruct((NC, N_GROUPS, EMB), _f32),
            jax.ShapeDtypeStruct((NC, N_GROUPS, 16), _f32),
        ),
        mesh=mesh,
        compiler_params=pltpu.CompilerParams(
            needs_layout_passes=False, use_tc_tiling_on_sc=False),
        scratch_types=[
            pltpu.MemorySpace.VMEM_SHARED((NS * UR, EMB), _f32),
            pltpu.MemorySpace.VMEM_SHARED((N_GROUPS, EMB), _f32),
            pltpu.MemorySpace.VMEM_SHARED((N_GROUPS, 16), _f32),
            pltpu.VMEM((80,), _i32),
            pltpu.VMEM((T,), _i32),
            pltpu.VMEM((T,), _f32),
            pltpu.VMEM((T,), _i32),
            pltpu.VMEM((T,), _i32),
            pltpu.VMEM((T, EMB), _f32),
            pltpu.VMEM((T,), _i32),
            pltpu.VMEM((T,), _i32),
            pltpu.VMEM((T, EMB), _f32),
            pltpu.VMEM((T, EMB), _f32),
            pltpu.VMEM((T,), _f32),
            pltpu.VMEM((T, EMB), _f32),
            pltpu.VMEM((T, 16), _f32),
            pltpu.VMEM((T, 16), _f32),
            pltpu.SemaphoreType.DMA,
        ],
    )
    partials, cnts = sc(starts, group_user_flat, user_seg_ids, behavior_items,
                        behavior_counts, behavior_user_ids, user_table,
                        item_table)

    out = pl.pallas_call(
        _combine_body,
        out_shape=jax.ShapeDtypeStruct((N_GROUPS, EMB), _f32),
    )(partials, cnts)
    return out


# speedup vs baseline: 4.5686x; 1.8314x over previous
import jax
import jax.numpy as jnp
from jax import lax
from jax.experimental import pallas as pl
from jax.experimental.pallas import tpu as pltpu
from jax.experimental.pallas import tpu_sc as plsc

N_GROUPS = 4096
TOTAL_USERS = 204800
TOTAL_BEHAVIORS = 2048000
EMB = 32

NC = 2
NS = 16
NW = NC * NS
UR = 1600
NRANGE = TOTAL_USERS // UR
ROUNDS = NRANGE // NW
T = 128
C = 64
NCHUNK = UR // C
GSLICE = N_GROUPS // NS
SPAD = 144

_i32 = jnp.int32
_f32 = jnp.float32


def _sread(ref, idx):
    return ref[pl.ds(idx, 16)][0]


def _sc_body(starts_h, gu_h, seg_h, bi_h, bc_h, bu_h, utab_h, itab_h,
             outp_h, outc_h,
             uacc_s, gacc_s, gcnt_s,
             starts_v,
             biA, bcA, buA, uofsA, rowsA,
             biB, bcB, buB, uofsB, rowsB,
             gu_v, seg_v, urows_v, acc_v, fm_v, zc_v, zc16_v, ones_v,
             semIA, semIB, semGA, semGB, semSA, semSB, semPB):
    c = lax.axis_index("c")
    s = lax.axis_index("s")
    wid = c * NS + s
    sbase = s * UR

    zero16 = jnp.zeros((16,), _f32)
    one16 = jnp.ones((16,), _f32)

    @pl.loop(0, T)
    def _fill(i):
        zc_v[i, pl.ds(0, 16)] = zero16
        zc_v[i, pl.ds(16, 16)] = zero16
        zc16_v[i, :] = zero16

    @pl.loop(0, C)
    def _fill1(i):
        ones_v[i, :] = one16

    for t in range(GSLICE // T):
        pltpu.sync_copy(zc_v, gacc_s.at[pl.ds(s * GSLICE + t * T, T)])
        pltpu.sync_copy(zc16_v, gcnt_s.at[pl.ds(s * GSLICE + t * T, T)])
    pltpu.sync_copy(starts_h, starts_v)
    plsc.subcore_barrier()

    def fetch_idx(base, bi, bc, bu, sem):
        pltpu.async_copy(bi_h.at[pl.ds(base, T)], bi, sem)
        pltpu.async_copy(bc_h.at[pl.ds(base, T)], bc, sem)
        pltpu.async_copy(bu_h.at[pl.ds(base, T)], bu, sem)

    def wait_idx(base, bi, bc, bu, sem):
        pltpu.make_async_copy(bi_h.at[pl.ds(base, T)], bi, sem).wait()
        pltpu.make_async_copy(bc_h.at[pl.ds(base, T)], bc, sem).wait()
        pltpu.make_async_copy(bu_h.at[pl.ds(base, T)], bu, sem).wait()

    def process(lo, bi, bc, bu, uofs, rows):
        for v8 in range(T // 16):
            sl = pl.ds(v8 * 16, 16)
            u = bu[sl] - lo
            valid = (u >= 0) & (u < UR)
            uofs[sl] = jnp.where(valid, u, 0) + sbase
            bc[sl] = jnp.where(valid, bc[sl], 0.0)

        @pl.loop(0, T, unroll=2)
        def _scale(rr):
            cidx = jnp.broadcast_to(rr, (16,))
            cv = plsc.load_gather(bc, [cidx])
            rows[rr, pl.ds(0, 16)] = rows[rr, pl.ds(0, 16)] * cv
            rows[rr, pl.ds(16, 16)] = rows[rr, pl.ds(16, 16)] * cv

    for j in range(ROUNDS):
        r = wid * ROUNDS + j
        lo = r * UR

        for t in range(UR // T):
            pltpu.sync_copy(zc_v, uacc_s.at[pl.ds(sbase + t * T, T)])
        if UR % T:
            rem = UR % T
            pltpu.sync_copy(zc_v.at[pl.ds(0, rem)],
                            uacc_s.at[pl.ds(sbase + (UR // T) * T, rem)])

        sA = _sread(starts_v, r)
        eA = _sread(starts_v, r + 1)
        k0 = sA >> 7
        k1 = (eA + (T - 1)) >> 7
        npair = (k1 - k0 + 1) >> 1

        @pl.loop(0, npair)
        def _pair(i2):
            ka = k0 + 2 * i2
            kb = ka + 1
            gb = kb < k1
            basea = ka * T
            baseb = kb * T

            fetch_idx(basea, biA, bcA, buA, semIA)

            @pl.when(gb)
            def _():
                fetch_idx(baseb, biB, bcB, buB, semIB)

            wait_idx(basea, biA, bcA, buA, semIA)
            pltpu.async_copy(itab_h.at[biA], rowsA, semGA)

            @pl.when(gb)
            def _():
                wait_idx(baseb, biB, bcB, buB, semIB)
                pltpu.async_copy(itab_h.at[biB], rowsB, semGB)

            pltpu.make_async_copy(itab_h.at[biA], rowsA, semGA).wait()
            process(lo, biA, bcA, buA, uofsA, rowsA)
            pltpu.async_copy(rowsA, uacc_s.at[uofsA], semSA, add=True)

            @pl.when(gb)
            def _():
                pltpu.make_async_copy(itab_h.at[biB], rowsB, semGB).wait()
                process(lo, biB, bcB, buB, uofsB, rowsB)
                pltpu.async_copy(rowsB, uacc_s.at[uofsB], semSB, add=True)

            pltpu.make_async_copy(rowsA, uacc_s.at[uofsA], semSA).wait()

            @pl.when(gb)
            def _():
                pltpu.make_async_copy(rowsB, uacc_s.at[uofsB], semSB).wait()

        @pl.loop(0, NCHUNK)
        def _chunk(q):
            ub = lo + q * C
            pltpu.sync_copy(gu_h.at[pl.ds(ub, C)], gu_v)
            pltpu.sync_copy(seg_h.at[pl.ds(ub, C)], seg_v)
            pltpu.async_copy(utab_h.at[gu_v], urows_v, semPB).wait()
            pltpu.sync_copy(uacc_s.at[pl.ds(sbase + q * C, C)], acc_v)
            for v8 in range(C // 16):
                sl = pl.ds(v8 * 16, 16)
                fm_v[sl] = jnp.where(gu_v[sl] != 0, 1.0, 0.0)

            @pl.loop(0, C, unroll=2)
            def _mul(rr):
                fidx = jnp.broadcast_to(rr, (16,))
                fv = plsc.load_gather(fm_v, [fidx])
                for h in (0, 16):
                    urows_v[rr, pl.ds(h, 16)] = (
                        urows_v[rr, pl.ds(h, 16)]
                        * acc_v[rr, pl.ds(h, 16)] * fv)

            pltpu.sync_copy(urows_v, gacc_s.at[seg_v], add=True)
            pltpu.sync_copy(ones_v, gcnt_s.at[seg_v], add=True)

    plsc.subcore_barrier()
    for t in range(GSLICE // T):
        off = s * GSLICE + t * T
        pltpu.sync_copy(gacc_s.at[pl.ds(off, T)], outp_h.at[c].at[pl.ds(off, T)])
        pltpu.sync_copy(gcnt_s.at[pl.ds(off, T)], outc_h.at[c].at[pl.ds(off, T)])


def _combine_body(p_ref, c_ref, o_ref):
    ps = p_ref[0] + p_ref[1]
    cnt = c_ref[0, :, 0:1] + c_ref[1, :, 0:1]
    o_ref[...] = ps / jnp.maximum(cnt, 1.0)


@jax.jit
def kernel(group_user_flat, user_seg_ids, behavior_items, behavior_counts,
           behavior_user_ids, user_table, item_table, lin_W, lin_b):
    del lin_W, lin_b
    bounds = jnp.arange(0, TOTAL_USERS + 1, UR, dtype=_i32)
    starts = jnp.searchsorted(behavior_user_ids, bounds).astype(_i32)
    starts = jnp.concatenate(
        [starts, jnp.full((SPAD - NRANGE - 1,), TOTAL_BEHAVIORS, _i32)])

    mesh = plsc.VectorSubcoreMesh(core_axis_name="c", subcore_axis_name="s")
    sc = pl.kernel(
        _sc_body,
        out_type=(
            jax.ShapeDtypeStruct((NC, N_GROUPS, EMB), _f32),
            jax.ShapeDtypeStruct((NC, N_GROUPS, 16), _f32),
        ),
        mesh=mesh,
        compiler_params=pltpu.CompilerParams(
            needs_layout_passes=False, use_tc_tiling_on_sc=False),
        scratch_types=[
            pltpu.MemorySpace.VMEM_SHARED((NS * UR, EMB), _f32),
            pltpu.MemorySpace.VMEM_SHARED((N_GROUPS, EMB), _f32),
            pltpu.MemorySpace.VMEM_SHARED((N_GROUPS, 16), _f32),
            pltpu.VMEM((SPAD,), _i32),
            pltpu.VMEM((T,), _i32),
            pltpu.VMEM((T,), _f32),
            pltpu.VMEM((T,), _i32),
            pltpu.VMEM((T,), _i32),
            pltpu.VMEM((T, EMB), _f32),
            pltpu.VMEM((T,), _i32),
            pltpu.VMEM((T,), _f32),
            pltpu.VMEM((T,), _i32),
            pltpu.VMEM((T,), _i32),
            pltpu.VMEM((T, EMB), _f32),
            pltpu.VMEM((C,), _i32),
            pltpu.VMEM((C,), _i32),
            pltpu.VMEM((C, EMB), _f32),
            pltpu.VMEM((C, EMB), _f32),
            pltpu.VMEM((C,), _f32),
            pltpu.VMEM((T, EMB), _f32),
            pltpu.VMEM((T, 16), _f32),
            pltpu.VMEM((C, 16), _f32),
            pltpu.SemaphoreType.DMA,
            pltpu.SemaphoreType.DMA,
            pltpu.SemaphoreType.DMA,
            pltpu.SemaphoreType.DMA,
            pltpu.SemaphoreType.DMA,
            pltpu.SemaphoreType.DMA,
            pltpu.SemaphoreType.DMA,
        ],
    )
    partials, cnts = sc(starts, group_user_flat, user_seg_ids, behavior_items,
                        behavior_counts, behavior_user_ids, user_table,
                        item_table)

    out = pl.pallas_call(
        _combine_body,
        out_shape=jax.ShapeDtypeStruct((N_GROUPS, EMB), _f32),
    )(partials, cnts)
    return out


# speedup vs baseline: 5.2387x; 1.1467x over previous
import jax
import jax.numpy as jnp
from jax import lax
from jax.experimental import pallas as pl
from jax.experimental.pallas import tpu as pltpu
from jax.experimental.pallas import tpu_sc as plsc

N_GROUPS = 4096
TOTAL_USERS = 204800
TOTAL_BEHAVIORS = 2048000
EMB = 32

NC = 2
NS = 16
NW = NC * NS
UR = 1600
NRANGE = TOTAL_USERS // UR
ROUNDS = NRANGE // NW
T = 128
WIDE = 4
C = 64
NCHUNK = UR // C
GSLICE = N_GROUPS // NS
SPAD = 144

_i32 = jnp.int32
_f32 = jnp.float32


def _sread(ref, idx):
    return ref[pl.ds(idx, 16)][0]


def _sc_body(*refs):
    (starts_h, gu_h, seg_h, bi_h, bc_h, bu_h, utab_h, itab_h,
     outp_h, outc_h, uacc_s, gacc_s, gcnt_s, starts_v) = refs[:14]
    p = 14
    bi = refs[p:p + WIDE]; p += WIDE
    bc = refs[p:p + WIDE]; p += WIDE
    bu = refs[p:p + WIDE]; p += WIDE
    uofs = refs[p:p + WIDE]; p += WIDE
    rows = refs[p:p + WIDE]; p += WIDE
    gu = refs[p:p + 2]; p += 2
    seg = refs[p:p + 2]; p += 2
    urows = refs[p:p + 2]; p += 2
    acc = refs[p:p + 2]; p += 2
    fm = refs[p:p + 2]; p += 2
    zc_v, zc16_v, ones_v = refs[p:p + 3]; p += 3
    semI = refs[p:p + WIDE]; p += WIDE
    semG = refs[p:p + WIDE]; p += WIDE
    semS = refs[p:p + WIDE]; p += WIDE
    semB = refs[p:p + 2]; p += 2
    semU = refs[p:p + 2]; p += 2
    semW = refs[p:p + 2]; p += 2

    c = lax.axis_index("c")
    s = lax.axis_index("s")
    wid = c * NS + s
    sbase = s * UR

    zero16 = jnp.zeros((16,), _f32)
    one16 = jnp.ones((16,), _f32)

    @pl.loop(0, T)
    def _fill(i):
        zc_v[i, pl.ds(0, 16)] = zero16
        zc_v[i, pl.ds(16, 16)] = zero16
        zc16_v[i, :] = zero16

    @pl.loop(0, C)
    def _fill1(i):
        ones_v[i, :] = one16

    for t in range(GSLICE // T):
        pltpu.sync_copy(zc_v, gacc_s.at[pl.ds(s * GSLICE + t * T, T)])
        pltpu.sync_copy(zc16_v, gcnt_s.at[pl.ds(s * GSLICE + t * T, T)])
    pltpu.sync_copy(starts_h, starts_v)
    plsc.subcore_barrier()

    def fetch_idx(base, x):
        pltpu.async_copy(bi_h.at[pl.ds(base, T)], bi[x], semI[x])
        pltpu.async_copy(bc_h.at[pl.ds(base, T)], bc[x], semI[x])
        pltpu.async_copy(bu_h.at[pl.ds(base, T)], bu[x], semI[x])

    def wait_idx(base, x):
        pltpu.make_async_copy(bi_h.at[pl.ds(base, T)], bi[x], semI[x]).wait()
        pltpu.make_async_copy(bc_h.at[pl.ds(base, T)], bc[x], semI[x]).wait()
        pltpu.make_async_copy(bu_h.at[pl.ds(base, T)], bu[x], semI[x]).wait()

    def process(lo, x):
        bcx, bux, uofsx, rowsx = bc[x], bu[x], uofs[x], rows[x]
        for v8 in range(T // 16):
            sl = pl.ds(v8 * 16, 16)
            u = bux[sl] - lo
            valid = (u >= 0) & (u < UR)
            uofsx[sl] = jnp.where(valid, u, 0) + sbase
            bcx[sl] = jnp.where(valid, bcx[sl], 0.0)

        @pl.loop(0, T, unroll=2)
        def _scale(rr):
            cidx = jnp.broadcast_to(rr, (16,))
            cv = plsc.load_gather(bcx, [cidx])
            rowsx[rr, pl.ds(0, 16)] = rowsx[rr, pl.ds(0, 16)] * cv
            rowsx[rr, pl.ds(16, 16)] = rowsx[rr, pl.ds(16, 16)] * cv

    for j in range(ROUNDS):
        r = wid * ROUNDS + j
        lo = r * UR

        for t in range(UR // T):
            pltpu.sync_copy(zc_v, uacc_s.at[pl.ds(sbase + t * T, T)])
        if UR % T:
            rem = UR % T
            pltpu.sync_copy(zc_v.at[pl.ds(0, rem)],
                            uacc_s.at[pl.ds(sbase + (UR // T) * T, rem)])

        sA = _sread(starts_v, r)
        eA = _sread(starts_v, r + 1)
        k0 = sA >> 7
        k1 = (eA + (T - 1)) >> 7
        ngroup = (k1 - k0 + (WIDE - 1)) >> 2

        @pl.loop(0, ngroup)
        def _group(ig):
            kx = [k0 + WIDE * ig + x for x in range(WIDE)]
            gx = [kx[x] < k1 for x in range(WIDE)]

            for x in range(WIDE):
                @pl.when(gx[x])
                def _(x=x):
                    fetch_idx(kx[x] * T, x)

            for x in range(WIDE):
                @pl.when(gx[x])
                def _(x=x):
                    wait_idx(kx[x] * T, x)
                    pltpu.async_copy(itab_h.at[bi[x]], rows[x], semG[x])

            for x in range(WIDE):
                @pl.when(gx[x])
                def _(x=x):
                    pltpu.make_async_copy(
                        itab_h.at[bi[x]], rows[x], semG[x]).wait()
                    process(lo, x)
                    pltpu.async_copy(rows[x], uacc_s.at[uofs[x]], semS[x],
                                     add=True)

            for x in range(WIDE):
                @pl.when(gx[x])
                def _(x=x):
                    pltpu.make_async_copy(
                        rows[x], uacc_s.at[uofs[x]], semS[x]).wait()

        def fetch_gs(q, y):
            ub = lo + q * C
            pltpu.async_copy(gu_h.at[pl.ds(ub, C)], gu[y], semB[y])
            pltpu.async_copy(seg_h.at[pl.ds(ub, C)], seg[y], semB[y])

        def wait_gs(q, y):
            ub = lo + q * C
            pltpu.make_async_copy(gu_h.at[pl.ds(ub, C)], gu[y], semB[y]).wait()
            pltpu.make_async_copy(seg_h.at[pl.ds(ub, C)], seg[y],
                                  semB[y]).wait()

        def process_b(q, y):
            guy, urowsy, accy, fmy = gu[y], urows[y], acc[y], fm[y]
            pltpu.sync_copy(uacc_s.at[pl.ds(sbase + q * C, C)], accy)
            for v8 in range(C // 16):
                sl = pl.ds(v8 * 16, 16)
                fmy[sl] = jnp.where(guy[sl] != 0, 1.0, 0.0)

            @pl.loop(0, C, unroll=2)
            def _mul(rr):
                fidx = jnp.broadcast_to(rr, (16,))
                fv = plsc.load_gather(fmy, [fidx])
                for h in (0, 16):
                    urowsy[rr, pl.ds(h, 16)] = (
                        urowsy[rr, pl.ds(h, 16)]
                        * accy[rr, pl.ds(h, 16)] * fv)

        @pl.loop(0, (NCHUNK + 1) // 2)
        def _bpair(i2):
            qa = 2 * i2
            qb = qa + 1
            gb = qb < NCHUNK

            fetch_gs(qa, 0)

            @pl.when(gb)
            def _():
                fetch_gs(qb, 1)

            wait_gs(qa, 0)
            pltpu.async_copy(utab_h.at[gu[0]], urows[0], semU[0])

            @pl.when(gb)
            def _():
                wait_gs(qb, 1)
                pltpu.async_copy(utab_h.at[gu[1]], urows[1], semU[1])

            pltpu.make_async_copy(utab_h.at[gu[0]], urows[0], semU[0]).wait()
            process_b(qa, 0)
            pltpu.async_copy(urows[0], gacc_s.at[seg[0]], semW[0], add=True)
            pltpu.async_copy(ones_v, gcnt_s.at[seg[0]], semW[0], add=True)

            @pl.when(gb)
            def _():
                pltpu.make_async_copy(
                    utab_h.at[gu[1]], urows[1], semU[1]).wait()
                process_b(qb, 1)
                pltpu.async_copy(urows[1], gacc_s.at[seg[1]], semW[1],
                                 add=True)
                pltpu.async_copy(ones_v, gcnt_s.at[seg[1]], semW[1], add=True)

            pltpu.make_async_copy(urows[0], gacc_s.at[seg[0]], semW[0]).wait()
            pltpu.make_async_copy(ones_v, gcnt_s.at[seg[0]], semW[0]).wait()

            @pl.when(gb)
            def _():
                pltpu.make_async_copy(
                    urows[1], gacc_s.at[seg[1]], semW[1]).wait()
                pltpu.make_async_copy(
                    ones_v, gcnt_s.at[seg[1]], semW[1]).wait()

    plsc.subcore_barrier()
    for t in range(GSLICE // T):
        off = s * GSLICE + t * T
        pltpu.sync_copy(gacc_s.at[pl.ds(off, T)], outp_h.at[c].at[pl.ds(off, T)])
        pltpu.sync_copy(gcnt_s.at[pl.ds(off, T)], outc_h.at[c].at[pl.ds(off, T)])


def _combine_body(p_ref, c_ref, o_ref):
    ps = p_ref[0] + p_ref[1]
    cnt = c_ref[0, :, 0:1] + c_ref[1, :, 0:1]
    o_ref[...] = ps / jnp.maximum(cnt, 1.0)


@jax.jit
def kernel(group_user_flat, user_seg_ids, behavior_items, behavior_counts,
           behavior_user_ids, user_table, item_table, lin_W, lin_b):
    del lin_W, lin_b
    bounds = jnp.arange(0, TOTAL_USERS + 1, UR, dtype=_i32)
    starts = jnp.searchsorted(behavior_user_ids, bounds).astype(_i32)
    starts = jnp.concatenate(
        [starts, jnp.full((SPAD - NRANGE - 1,), TOTAL_BEHAVIORS, _i32)])

    mesh = plsc.VectorSubcoreMesh(core_axis_name="c", subcore_axis_name="s")
    scratch = [
        pltpu.MemorySpace.VMEM_SHARED((NS * UR, EMB), _f32),
        pltpu.MemorySpace.VMEM_SHARED((N_GROUPS, EMB), _f32),
        pltpu.MemorySpace.VMEM_SHARED((N_GROUPS, 16), _f32),
        pltpu.VMEM((SPAD,), _i32),
    ]
    scratch += [pltpu.VMEM((T,), _i32) for _ in range(WIDE)]
    scratch += [pltpu.VMEM((T,), _f32) for _ in range(WIDE)]
    scratch += [pltpu.VMEM((T,), _i32) for _ in range(WIDE)]
    scratch += [pltpu.VMEM((T,), _i32) for _ in range(WIDE)]
    scratch += [pltpu.VMEM((T, EMB), _f32) for _ in range(WIDE)]
    scratch += [pltpu.VMEM((C,), _i32) for _ in range(2)]
    scratch += [pltpu.VMEM((C,), _i32) for _ in range(2)]
    scratch += [pltpu.VMEM((C, EMB), _f32) for _ in range(2)]
    scratch += [pltpu.VMEM((C, EMB), _f32) for _ in range(2)]
    scratch += [pltpu.VMEM((C,), _f32) for _ in range(2)]
    scratch += [
        pltpu.VMEM((T, EMB), _f32),
        pltpu.VMEM((T, 16), _f32),
        pltpu.VMEM((C, 16), _f32),
    ]
    scratch += [pltpu.SemaphoreType.DMA] * (3 * WIDE + 6)

    sc = pl.kernel(
        _sc_body,
        out_type=(
            jax.ShapeDtypeStruct((NC, N_GROUPS, EMB), _f32),
            jax.ShapeDtypeStruct((NC, N_GROUPS, 16), _f32),
        ),
        mesh=mesh,
        compiler_params=pltpu.CompilerParams(
            needs_layout_passes=False, use_tc_tiling_on_sc=False),
        scratch_types=scratch,
    )
    partials, cnts = sc(starts, group_user_flat, user_seg_ids, behavior_items,
                        behavior_counts, behavior_user_ids, user_table,
                        item_table)

    out = pl.pallas_call(
        _combine_body,
        out_shape=jax.ShapeDtypeStruct((N_GROUPS, EMB), _f32),
    )(partials, cnts)
    return out


# speedup vs baseline: 5.3311x; 1.0176x over previous
import jax
import jax.numpy as jnp
from jax import lax
from jax.experimental import pallas as pl
from jax.experimental.pallas import tpu as pltpu
from jax.experimental.pallas import tpu_sc as plsc

N_GROUPS = 4096
TOTAL_USERS = 204800
TOTAL_BEHAVIORS = 2048000
EMB = 32

NC = 2
NS = 16
NW = NC * NS
UR = 1600
NRANGE = TOTAL_USERS // UR
ROUNDS = NRANGE // NW
T = 128
WIDE = 4
C = 64
NCHUNK = UR // C
GSLICE = N_GROUPS // NS
SPAD = 144

_i32 = jnp.int32
_f32 = jnp.float32


def _sread(ref, idx):
    return ref[pl.ds(idx, 16)][0]


def _sc_body(*refs):
    (starts_h, gu_h, seg_h, bi_h, bc_h, bu_h, utab_h, itab_h,
     outp_h, outc_h, uacc_s, gacc_s, gcnt_s, starts_v) = refs[:14]
    p = 14
    bi = refs[p:p + WIDE]; p += WIDE
    bc = refs[p:p + WIDE]; p += WIDE
    bu = refs[p:p + WIDE]; p += WIDE
    uofs = refs[p:p + WIDE]; p += WIDE
    rows = refs[p:p + WIDE]; p += WIDE
    gu = refs[p:p + 2]; p += 2
    seg = refs[p:p + 2]; p += 2
    urows = refs[p:p + 2]; p += 2
    acc = refs[p:p + 2]; p += 2
    fm = refs[p:p + 2]; p += 2
    zc_v, zc16_v, ones_v = refs[p:p + 3]; p += 3
    semI = refs[p:p + WIDE]; p += WIDE
    semG = refs[p:p + WIDE]; p += WIDE
    semS = refs[p:p + WIDE]; p += WIDE
    semB = refs[p:p + 2]; p += 2
    semU = refs[p:p + 2]; p += 2
    semW = refs[p:p + 2]; p += 2

    c = lax.axis_index("c")
    s = lax.axis_index("s")
    wid = c * NS + s
    sbase = s * UR

    zero16 = jnp.zeros((16,), _f32)
    one16 = jnp.ones((16,), _f32)

    @pl.loop(0, T)
    def _fill(i):
        zc_v[i, pl.ds(0, 16)] = zero16
        zc_v[i, pl.ds(16, 16)] = zero16
        zc16_v[i, :] = zero16

    @pl.loop(0, C)
    def _fill1(i):
        ones_v[i, :] = one16

    for t in range(GSLICE // T):
        pltpu.sync_copy(zc_v, gacc_s.at[pl.ds(s * GSLICE + t * T, T)])
        pltpu.sync_copy(zc16_v, gcnt_s.at[pl.ds(s * GSLICE + t * T, T)])
    pltpu.sync_copy(starts_h, starts_v)
    plsc.subcore_barrier()

    def fetch_idx(base, x):
        pltpu.async_copy(bi_h.at[pl.ds(base, T)], bi[x], semI[x])
        pltpu.async_copy(bc_h.at[pl.ds(base, T)], bc[x], semI[x])
        pltpu.async_copy(bu_h.at[pl.ds(base, T)], bu[x], semI[x])

    def wait_idx(base, x):
        pltpu.make_async_copy(bi_h.at[pl.ds(base, T)], bi[x], semI[x]).wait()
        pltpu.make_async_copy(bc_h.at[pl.ds(base, T)], bc[x], semI[x]).wait()
        pltpu.make_async_copy(bu_h.at[pl.ds(base, T)], bu[x], semI[x]).wait()

    def process(lo, x):
        bcx, bux, uofsx, rowsx = bc[x], bu[x], uofs[x], rows[x]
        for v8 in range(T // 16):
            sl = pl.ds(v8 * 16, 16)
            u = bux[sl] - lo
            valid = (u >= 0) & (u < UR)
            uofsx[sl] = jnp.where(valid, u, 0) + sbase
            bcx[sl] = jnp.where(valid, bcx[sl], 0.0)

        @pl.loop(0, T, unroll=8)
        def _scale(rr):
            cidx = jnp.broadcast_to(rr, (16,))
            cv = plsc.load_gather(bcx, [cidx])
            rowsx[rr, pl.ds(0, 16)] = rowsx[rr, pl.ds(0, 16)] * cv
            rowsx[rr, pl.ds(16, 16)] = rowsx[rr, pl.ds(16, 16)] * cv

    for j in range(ROUNDS):
        r = wid * ROUNDS + j
        lo = r * UR

        for t in range(UR // T):
            pltpu.sync_copy(zc_v, uacc_s.at[pl.ds(sbase + t * T, T)])
        if UR % T:
            rem = UR % T
            pltpu.sync_copy(zc_v.at[pl.ds(0, rem)],
                            uacc_s.at[pl.ds(sbase + (UR // T) * T, rem)])

        sA = _sread(starts_v, r)
        eA = _sread(starts_v, r + 1)
        k0 = sA >> 7
        k1 = (eA + (T - 1)) >> 7
        ngroup = (k1 - k0 + (WIDE - 1)) >> 2

        @pl.loop(0, ngroup)
        def _group(ig):
            kx = [k0 + WIDE * ig + x for x in range(WIDE)]
            gx = [kx[x] < k1 for x in range(WIDE)]

            for x in range(WIDE):
                @pl.when(gx[x])
                def _(x=x):
                    fetch_idx(kx[x] * T, x)

            for x in range(WIDE):
                @pl.when(gx[x])
                def _(x=x):
                    wait_idx(kx[x] * T, x)
                    pltpu.async_copy(itab_h.at[bi[x]], rows[x], semG[x])

            for x in range(WIDE):
                @pl.when(gx[x])
                def _(x=x):
                    pltpu.make_async_copy(
                        itab_h.at[bi[x]], rows[x], semG[x]).wait()
                    process(lo, x)
                    pltpu.async_copy(rows[x], uacc_s.at[uofs[x]], semS[x],
                                     add=True)

            for x in range(WIDE):
                @pl.when(gx[x])
                def _(x=x):
                    pltpu.make_async_copy(
                        rows[x], uacc_s.at[uofs[x]], semS[x]).wait()

        def fetch_gs(q, y):
            ub = lo + q * C
            pltpu.async_copy(gu_h.at[pl.ds(ub, C)], gu[y], semB[y])
            pltpu.async_copy(seg_h.at[pl.ds(ub, C)], seg[y], semB[y])

        def wait_gs(q, y):
            ub = lo + q * C
            pltpu.make_async_copy(gu_h.at[pl.ds(ub, C)], gu[y], semB[y]).wait()
            pltpu.make_async_copy(seg_h.at[pl.ds(ub, C)], seg[y],
                                  semB[y]).wait()

        def process_b(q, y):
            guy, urowsy, accy, fmy = gu[y], urows[y], acc[y], fm[y]
            pltpu.sync_copy(uacc_s.at[pl.ds(sbase + q * C, C)], accy)
            for v8 in range(C // 16):
                sl = pl.ds(v8 * 16, 16)
                fmy[sl] = jnp.where(guy[sl] != 0, 1.0, 0.0)

            @pl.loop(0, C, unroll=8)
            def _mul(rr):
                fidx = jnp.broadcast_to(rr, (16,))
                fv = plsc.load_gather(fmy, [fidx])
                for h in (0, 16):
                    urowsy[rr, pl.ds(h, 16)] = (
                        urowsy[rr, pl.ds(h, 16)]
                        * accy[rr, pl.ds(h, 16)] * fv)

        @pl.loop(0, (NCHUNK + 1) // 2)
        def _bpair(i2):
            qa = 2 * i2
            qb = qa + 1
            gb = qb < NCHUNK

            fetch_gs(qa, 0)

            @pl.when(gb)
            def _():
                fetch_gs(qb, 1)

            wait_gs(qa, 0)
            pltpu.async_copy(utab_h.at[gu[0]], urows[0], semU[0])

            @pl.when(gb)
            def _():
                wait_gs(qb, 1)
                pltpu.async_copy(utab_h.at[gu[1]], urows[1], semU[1])

            pltpu.make_async_copy(utab_h.at[gu[0]], urows[0], semU[0]).wait()
            process_b(qa, 0)
            pltpu.async_copy(urows[0], gacc_s.at[seg[0]], semW[0], add=True)
            pltpu.async_copy(ones_v, gcnt_s.at[seg[0]], semW[0], add=True)

            @pl.when(gb)
            def _():
                pltpu.make_async_copy(
                    utab_h.at[gu[1]], urows[1], semU[1]).wait()
                process_b(qb, 1)
                pltpu.async_copy(urows[1], gacc_s.at[seg[1]], semW[1],
                                 add=True)
                pltpu.async_copy(ones_v, gcnt_s.at[seg[1]], semW[1], add=True)

            pltpu.make_async_copy(urows[0], gacc_s.at[seg[0]], semW[0]).wait()
            pltpu.make_async_copy(ones_v, gcnt_s.at[seg[0]], semW[0]).wait()

            @pl.when(gb)
            def _():
                pltpu.make_async_copy(
                    urows[1], gacc_s.at[seg[1]], semW[1]).wait()
                pltpu.make_async_copy(
                    ones_v, gcnt_s.at[seg[1]], semW[1]).wait()

    plsc.subcore_barrier()
    for t in range(GSLICE // T):
        off = s * GSLICE + t * T
        pltpu.sync_copy(gacc_s.at[pl.ds(off, T)], outp_h.at[c].at[pl.ds(off, T)])
        pltpu.sync_copy(gcnt_s.at[pl.ds(off, T)], outc_h.at[c].at[pl.ds(off, T)])


def _combine_body(p_ref, c_ref, o_ref):
    ps = p_ref[0] + p_ref[1]
    cnt = c_ref[0, :, 0:1] + c_ref[1, :, 0:1]
    o_ref[...] = ps / jnp.maximum(cnt, 1.0)


@jax.jit
def kernel(group_user_flat, user_seg_ids, behavior_items, behavior_counts,
           behavior_user_ids, user_table, item_table, lin_W, lin_b):
    del lin_W, lin_b
    bounds = jnp.arange(0, TOTAL_USERS + 1, UR, dtype=_i32)
    starts = jnp.searchsorted(behavior_user_ids, bounds).astype(_i32)
    starts = jnp.concatenate(
        [starts, jnp.full((SPAD - NRANGE - 1,), TOTAL_BEHAVIORS, _i32)])

    mesh = plsc.VectorSubcoreMesh(core_axis_name="c", subcore_axis_name="s")
    scratch = [
        pltpu.MemorySpace.VMEM_SHARED((NS * UR, EMB), _f32),
        pltpu.MemorySpace.VMEM_SHARED((N_GROUPS, EMB), _f32),
        pltpu.MemorySpace.VMEM_SHARED((N_GROUPS, 16), _f32),
        pltpu.VMEM((SPAD,), _i32),
    ]
    scratch += [pltpu.VMEM((T,), _i32) for _ in range(WIDE)]
    scratch += [pltpu.VMEM((T,), _f32) for _ in range(WIDE)]
    scratch += [pltpu.VMEM((T,), _i32) for _ in range(WIDE)]
    scratch += [pltpu.VMEM((T,), _i32) for _ in range(WIDE)]
    scratch += [pltpu.VMEM((T, EMB), _f32) for _ in range(WIDE)]
    scratch += [pltpu.VMEM((C,), _i32) for _ in range(2)]
    scratch += [pltpu.VMEM((C,), _i32) for _ in range(2)]
    scratch += [pltpu.VMEM((C, EMB), _f32) for _ in range(2)]
    scratch += [pltpu.VMEM((C, EMB), _f32) for _ in range(2)]
    scratch += [pltpu.VMEM((C,), _f32) for _ in range(2)]
    scratch += [
        pltpu.VMEM((T, EMB), _f32),
        pltpu.VMEM((T, 16), _f32),
        pltpu.VMEM((C, 16), _f32),
    ]
    scratch += [pltpu.SemaphoreType.DMA] * (3 * WIDE + 6)

    sc = pl.kernel(
        _sc_body,
        out_type=(
            jax.ShapeDtypeStruct((NC, N_GROUPS, EMB), _f32),
            jax.ShapeDtypeStruct((NC, N_GROUPS, 16), _f32),
        ),
        mesh=mesh,
        compiler_params=pltpu.CompilerParams(
            needs_layout_passes=False, use_tc_tiling_on_sc=False),
        scratch_types=scratch,
    )
    partials, cnts = sc(starts, group_user_flat, user_seg_ids, behavior_items,
                        behavior_counts, behavior_user_ids, user_table,
                        item_table)

    out = pl.pallas_call(
        _combine_body,
        out_shape=jax.ShapeDtypeStruct((N_GROUPS, EMB), _f32),
    )(partials, cnts)
    return out


# speedup vs baseline: 5.4016x; 1.0132x over previous
import jax
import jax.numpy as jnp
from jax import lax
from jax.experimental import pallas as pl
from jax.experimental.pallas import tpu as pltpu
from jax.experimental.pallas import tpu_sc as plsc

N_GROUPS = 4096
TOTAL_USERS = 204800
TOTAL_BEHAVIORS = 2048000
EMB = 32

NC = 2
NS = 16
NW = NC * NS
UR = 1600
NRANGE = TOTAL_USERS // UR
ROUNDS = NRANGE // NW
T = 128
WIDE = 4
C = 64
NCHUNK = UR // C
GSLICE = N_GROUPS // NS
SPAD = 144

_i32 = jnp.int32
_f32 = jnp.float32


def _sread(ref, idx):
    return ref[pl.ds(idx, 16)][0]


def _sc_body(*refs):
    (starts_h, gu_h, seg_h, bi_h, bc_h, bu_h, utab_h, itab_h,
     outp_h, outc_h, uacc_s, gacc_s, gcnt_s, starts_v) = refs[:14]
    p = 14
    bi = refs[p:p + WIDE]; p += WIDE
    bc = refs[p:p + WIDE]; p += WIDE
    bu = refs[p:p + WIDE]; p += WIDE
    uofs = refs[p:p + WIDE]; p += WIDE
    rows = refs[p:p + WIDE]; p += WIDE
    gu = refs[p:p + 2]; p += 2
    seg = refs[p:p + 2]; p += 2
    urows = refs[p:p + 2]; p += 2
    acc = refs[p:p + 2]; p += 2
    fm = refs[p:p + 2]; p += 2
    zc_v, zc16_v, ones_v = refs[p:p + 3]; p += 3
    semI = refs[p:p + WIDE]; p += WIDE
    semG = refs[p:p + WIDE]; p += WIDE
    semS = refs[p:p + WIDE]; p += WIDE
    semB = refs[p:p + 2]; p += 2
    semU = refs[p:p + 2]; p += 2
    semW = refs[p:p + 2]; p += 2

    c = lax.axis_index("c")
    s = lax.axis_index("s")
    wid = c * NS + s
    sbase = s * UR

    zero16 = jnp.zeros((16,), _f32)
    one16 = jnp.ones((16,), _f32)

    @pl.loop(0, T)
    def _fill(i):
        zc_v[i, pl.ds(0, 16)] = zero16
        zc_v[i, pl.ds(16, 16)] = zero16
        zc16_v[i, :] = zero16

    @pl.loop(0, C)
    def _fill1(i):
        ones_v[i, :] = one16

    for t in range(GSLICE // T):
        pltpu.sync_copy(zc_v, gacc_s.at[pl.ds(s * GSLICE + t * T, T)])
        pltpu.sync_copy(zc16_v, gcnt_s.at[pl.ds(s * GSLICE + t * T, T)])
    pltpu.sync_copy(starts_h, starts_v)
    plsc.subcore_barrier()

    def fetch_idx(base, x):
        pltpu.async_copy(bi_h.at[pl.ds(base, T)], bi[x], semI[x])
        pltpu.async_copy(bc_h.at[pl.ds(base, T)], bc[x], semI[x])
        pltpu.async_copy(bu_h.at[pl.ds(base, T)], bu[x], semI[x])

    def wait_idx(base, x):
        pltpu.make_async_copy(bi_h.at[pl.ds(base, T)], bi[x], semI[x]).wait()
        pltpu.make_async_copy(bc_h.at[pl.ds(base, T)], bc[x], semI[x]).wait()
        pltpu.make_async_copy(bu_h.at[pl.ds(base, T)], bu[x], semI[x]).wait()

    def process(lo, x):
        bcx, bux, uofsx, rowsx = bc[x], bu[x], uofs[x], rows[x]
        for v8 in range(T // 16):
            sl = pl.ds(v8 * 16, 16)
            u = bux[sl] - lo
            valid = (u >= 0) & (u < UR)
            uofsx[sl] = jnp.where(valid, u, 0) + sbase
            bcx[sl] = jnp.where(valid, bcx[sl], 0.0)

        @pl.loop(0, T, unroll=8)
        def _scale(rr):
            cidx = jnp.broadcast_to(rr, (16,))
            cv = plsc.load_gather(bcx, [cidx])
            rowsx[rr, pl.ds(0, 16)] = rowsx[rr, pl.ds(0, 16)] * cv
            rowsx[rr, pl.ds(16, 16)] = rowsx[rr, pl.ds(16, 16)] * cv

    for j in range(ROUNDS):
        r = wid * ROUNDS + j
        lo = r * UR

        for t in range(UR // T):
            pltpu.sync_copy(zc_v, uacc_s.at[pl.ds(sbase + t * T, T)])
        if UR % T:
            rem = UR % T
            pltpu.sync_copy(zc_v.at[pl.ds(0, rem)],
                            uacc_s.at[pl.ds(sbase + (UR // T) * T, rem)])

        sA = _sread(starts_v, r)
        eA = _sread(starts_v, r + 1)
        k0 = sA >> 7
        k1 = (eA + (T - 1)) >> 7
        ngroup = (k1 - k0 + (WIDE - 1)) >> 2

        for x in range(WIDE):
            @pl.when(k0 + x < k1)
            def _(x=x):
                fetch_idx((k0 + x) * T, x)
        for x in range(WIDE):
            @pl.when(k0 + x < k1)
            def _(x=x):
                wait_idx((k0 + x) * T, x)
                pltpu.async_copy(itab_h.at[bi[x]], rows[x], semG[x])

        @pl.loop(0, ngroup)
        def _group(ig):
            kx = [k0 + WIDE * ig + x for x in range(WIDE)]
            nx = [kx[x] + WIDE for x in range(WIDE)]
            gx = [kx[x] < k1 for x in range(WIDE)]
            hx = [nx[x] < k1 for x in range(WIDE)]

            for x in range(WIDE):
                @pl.when(gx[x])
                def _(x=x):
                    pltpu.make_async_copy(
                        itab_h.at[bi[x]], rows[x], semG[x]).wait()
                    process(lo, x)
                    pltpu.async_copy(rows[x], uacc_s.at[uofs[x]], semS[x],
                                     add=True)

            for x in range(WIDE):
                @pl.when(hx[x])
                def _(x=x):
                    fetch_idx(nx[x] * T, x)

            for x in range(WIDE):
                @pl.when(hx[x])
                def _(x=x):
                    wait_idx(nx[x] * T, x)
                    pltpu.make_async_copy(
                        rows[x], uacc_s.at[uofs[x]], semS[x]).wait()
                    pltpu.async_copy(itab_h.at[bi[x]], rows[x], semG[x])

            for x in range(WIDE):
                @pl.when(gx[x] & jnp.logical_not(hx[x]))
                def _(x=x):
                    pltpu.make_async_copy(
                        rows[x], uacc_s.at[uofs[x]], semS[x]).wait()

        def fetch_gs(q, y):
            ub = lo + q * C
            pltpu.async_copy(gu_h.at[pl.ds(ub, C)], gu[y], semB[y])
            pltpu.async_copy(seg_h.at[pl.ds(ub, C)], seg[y], semB[y])

        def wait_gs(q, y):
            ub = lo + q * C
            pltpu.make_async_copy(gu_h.at[pl.ds(ub, C)], gu[y], semB[y]).wait()
            pltpu.make_async_copy(seg_h.at[pl.ds(ub, C)], seg[y],
                                  semB[y]).wait()

        def process_b(q, y):
            guy, urowsy, accy, fmy = gu[y], urows[y], acc[y], fm[y]
            pltpu.sync_copy(uacc_s.at[pl.ds(sbase + q * C, C)], accy)
            for v8 in range(C // 16):
                sl = pl.ds(v8 * 16, 16)
                fmy[sl] = jnp.where(guy[sl] != 0, 1.0, 0.0)

            @pl.loop(0, C, unroll=8)
            def _mul(rr):
                fidx = jnp.broadcast_to(rr, (16,))
                fv = plsc.load_gather(fmy, [fidx])
                for h in (0, 16):
                    urowsy[rr, pl.ds(h, 16)] = (
                        urowsy[rr, pl.ds(h, 16)]
                        * accy[rr, pl.ds(h, 16)] * fv)

        @pl.loop(0, (NCHUNK + 1) // 2)
        def _bpair(i2):
            qa = 2 * i2
            qb = qa + 1
            gb = qb < NCHUNK

            fetch_gs(qa, 0)

            @pl.when(gb)
            def _():
                fetch_gs(qb, 1)

            wait_gs(qa, 0)
            pltpu.async_copy(utab_h.at[gu[0]], urows[0], semU[0])

            @pl.when(gb)
            def _():
                wait_gs(qb, 1)
                pltpu.async_copy(utab_h.at[gu[1]], urows[1], semU[1])

            pltpu.make_async_copy(utab_h.at[gu[0]], urows[0], semU[0]).wait()
            process_b(qa, 0)
            pltpu.async_copy(urows[0], gacc_s.at[seg[0]], semW[0], add=True)
            pltpu.async_copy(ones_v, gcnt_s.at[seg[0]], semW[0], add=True)

            @pl.when(gb)
            def _():
                pltpu.make_async_copy(
                    utab_h.at[gu[1]], urows[1], semU[1]).wait()
                process_b(qb, 1)
                pltpu.async_copy(urows[1], gacc_s.at[seg[1]], semW[1],
                                 add=True)
                pltpu.async_copy(ones_v, gcnt_s.at[seg[1]], semW[1], add=True)

            pltpu.make_async_copy(urows[0], gacc_s.at[seg[0]], semW[0]).wait()
            pltpu.make_async_copy(ones_v, gcnt_s.at[seg[0]], semW[0]).wait()

            @pl.when(gb)
            def _():
                pltpu.make_async_copy(
                    urows[1], gacc_s.at[seg[1]], semW[1]).wait()
                pltpu.make_async_copy(
                    ones_v, gcnt_s.at[seg[1]], semW[1]).wait()

    plsc.subcore_barrier()
    for t in range(GSLICE // T):
        off = s * GSLICE + t * T
        pltpu.sync_copy(gacc_s.at[pl.ds(off, T)], outp_h.at[c].at[pl.ds(off, T)])
        pltpu.sync_copy(gcnt_s.at[pl.ds(off, T)], outc_h.at[c].at[pl.ds(off, T)])


def _combine_body(p_ref, c_ref, o_ref):
    ps = p_ref[0] + p_ref[1]
    cnt = c_ref[0, :, 0:1] + c_ref[1, :, 0:1]
    o_ref[...] = ps / jnp.maximum(cnt, 1.0)


@jax.jit
def kernel(group_user_flat, user_seg_ids, behavior_items, behavior_counts,
           behavior_user_ids, user_table, item_table, lin_W, lin_b):
    del lin_W, lin_b
    bounds = jnp.arange(0, TOTAL_USERS + 1, UR, dtype=_i32)
    starts = jnp.searchsorted(behavior_user_ids, bounds).astype(_i32)
    starts = jnp.concatenate(
        [starts, jnp.full((SPAD - NRANGE - 1,), TOTAL_BEHAVIORS, _i32)])

    mesh = plsc.VectorSubcoreMesh(core_axis_name="c", subcore_axis_name="s")
    scratch = [
        pltpu.MemorySpace.VMEM_SHARED((NS * UR, EMB), _f32),
        pltpu.MemorySpace.VMEM_SHARED((N_GROUPS, EMB), _f32),
        pltpu.MemorySpace.VMEM_SHARED((N_GROUPS, 16), _f32),
        pltpu.VMEM((SPAD,), _i32),
    ]
    scratch += [pltpu.VMEM((T,), _i32) for _ in range(WIDE)]
    scratch += [pltpu.VMEM((T,), _f32) for _ in range(WIDE)]
    scratch += [pltpu.VMEM((T,), _i32) for _ in range(WIDE)]
    scratch += [pltpu.VMEM((T,), _i32) for _ in range(WIDE)]
    scratch += [pltpu.VMEM((T, EMB), _f32) for _ in range(WIDE)]
    scratch += [pltpu.VMEM((C,), _i32) for _ in range(2)]
    scratch += [pltpu.VMEM((C,), _i32) for _ in range(2)]
    scratch += [pltpu.VMEM((C, EMB), _f32) for _ in range(2)]
    scratch += [pltpu.VMEM((C, EMB), _f32) for _ in range(2)]
    scratch += [pltpu.VMEM((C,), _f32) for _ in range(2)]
    scratch += [
        pltpu.VMEM((T, EMB), _f32),
        pltpu.VMEM((T, 16), _f32),
        pltpu.VMEM((C, 16), _f32),
    ]
    scratch += [pltpu.SemaphoreType.DMA] * (3 * WIDE + 6)

    sc = pl.kernel(
        _sc_body,
        out_type=(
            jax.ShapeDtypeStruct((NC, N_GROUPS, EMB), _f32),
            jax.ShapeDtypeStruct((NC, N_GROUPS, 16), _f32),
        ),
        mesh=mesh,
        compiler_params=pltpu.CompilerParams(
            needs_layout_passes=False, use_tc_tiling_on_sc=False),
        scratch_types=scratch,
    )
    partials, cnts = sc(starts, group_user_flat, user_seg_ids, behavior_items,
                        behavior_counts, behavior_user_ids, user_table,
                        item_table)

    out = pl.pallas_call(
        _combine_body,
        out_shape=jax.ShapeDtypeStruct((N_GROUPS, EMB), _f32),
    )(partials, cnts)
    return out


# speedup vs baseline: 6.6300x; 1.2274x over previous
import jax
import jax.numpy as jnp
from jax import lax
from jax.experimental import pallas as pl
from jax.experimental.pallas import tpu as pltpu
from jax.experimental.pallas import tpu_sc as plsc

N_GROUPS = 4096
TOTAL_USERS = 204800
TOTAL_BEHAVIORS = 2048000
EMB = 32

NC = 2
NS = 16
NW = NC * NS
UR = 1600
NRANGE = TOTAL_USERS // UR
ROUNDS = NRANGE // NW
T = 128
WIDE = 4
C = 64
NCHUNK = UR // C
GSLICE = N_GROUPS // NS
SPAD = 144

_i32 = jnp.int32
_f32 = jnp.float32


def _sread(ref, idx):
    return ref[pl.ds(idx, 16)][0]


def _sc_body(*refs):
    (starts_h, gu_h, seg_h, bi_h, bc_h, bu_h, utab_h, itab_h,
     outp_h, outc_h, uacc_s, gacc_s, gcnt_s, starts_v) = refs[:14]
    p = 14
    bi = refs[p:p + WIDE]; p += WIDE
    bc = refs[p:p + WIDE]; p += WIDE
    bu = refs[p:p + WIDE]; p += WIDE
    uofs = refs[p:p + WIDE]; p += WIDE
    rows = refs[p:p + WIDE]; p += WIDE
    gu = refs[p:p + 2]; p += 2
    seg = refs[p:p + 2]; p += 2
    urows = refs[p:p + 2]; p += 2
    acc = refs[p:p + 2]; p += 2
    fm = refs[p:p + 2]; p += 2
    zc_v, zc16_v, ones_v = refs[p:p + 3]; p += 3
    semI = refs[p:p + WIDE]; p += WIDE
    semG = refs[p:p + WIDE]; p += WIDE
    semS = refs[p:p + WIDE]; p += WIDE
    semB = refs[p:p + 2]; p += 2
    semU = refs[p:p + 2]; p += 2
    semW = refs[p:p + 2]; p += 2

    c = lax.axis_index("c")
    s = lax.axis_index("s")
    wid = c * NS + s
    sbase = s * UR

    zero16 = jnp.zeros((16,), _f32)
    one16 = jnp.ones((16,), _f32)

    @pl.loop(0, T)
    def _fill(i):
        zc_v[i, pl.ds(0, 16)] = zero16
        zc_v[i, pl.ds(16, 16)] = zero16
        zc16_v[i, :] = zero16

    @pl.loop(0, C)
    def _fill1(i):
        ones_v[i, :] = one16

    for t in range(GSLICE // T):
        pltpu.sync_copy(zc_v, gacc_s.at[pl.ds(s * GSLICE + t * T, T)])
        pltpu.sync_copy(zc16_v, gcnt_s.at[pl.ds(s * GSLICE + t * T, T)])
    pltpu.sync_copy(starts_h, starts_v)
    plsc.subcore_barrier()

    def fetch_idx(base, x):
        pltpu.async_copy(bi_h.at[pl.ds(base, T)], bi[x], semI[x])
        pltpu.async_copy(bc_h.at[pl.ds(base, T)], bc[x], semI[x])
        pltpu.async_copy(bu_h.at[pl.ds(base, T)], bu[x], semI[x])

    def wait_idx(base, x):
        pltpu.make_async_copy(bi_h.at[pl.ds(base, T)], bi[x], semI[x]).wait()
        pltpu.make_async_copy(bc_h.at[pl.ds(base, T)], bc[x], semI[x]).wait()
        pltpu.make_async_copy(bu_h.at[pl.ds(base, T)], bu[x], semI[x]).wait()

    def process(lo, x):
        bcx, bux, uofsx, rowsx = bc[x], bu[x], uofs[x], rows[x]
        for v8 in range(T // 16):
            sl = pl.ds(v8 * 16, 16)
            u = bux[sl] - lo
            valid = (u >= 0) & (u < UR)
            uofsx[sl] = jnp.where(valid, u, 0) + sbase
            bcx[sl] = jnp.where(valid, bcx[sl], 0.0)

        @pl.loop(0, T // 16)
        def _scale(b):
            base = b * 16
            cnt16 = bcx[pl.ds(base, 16)]
            for rsub in range(16):
                cv = jnp.broadcast_to(cnt16[rsub], (16,))
                rr = base + rsub
                rowsx[rr, pl.ds(0, 16)] = rowsx[rr, pl.ds(0, 16)] * cv
                rowsx[rr, pl.ds(16, 16)] = rowsx[rr, pl.ds(16, 16)] * cv

    for j in range(ROUNDS):
        r = wid * ROUNDS + j
        lo = r * UR

        for t in range(UR // T):
            pltpu.sync_copy(zc_v, uacc_s.at[pl.ds(sbase + t * T, T)])
        if UR % T:
            rem = UR % T
            pltpu.sync_copy(zc_v.at[pl.ds(0, rem)],
                            uacc_s.at[pl.ds(sbase + (UR // T) * T, rem)])

        sA = _sread(starts_v, r)
        eA = _sread(starts_v, r + 1)
        k0 = sA >> 7
        k1 = (eA + (T - 1)) >> 7
        ngroup = (k1 - k0 + (WIDE - 1)) >> 2

        for x in range(WIDE):
            @pl.when(k0 + x < k1)
            def _(x=x):
                fetch_idx((k0 + x) * T, x)
        for x in range(WIDE):
            @pl.when(k0 + x < k1)
            def _(x=x):
                wait_idx((k0 + x) * T, x)
                pltpu.async_copy(itab_h.at[bi[x]], rows[x], semG[x])

        @pl.loop(0, ngroup)
        def _group(ig):
            kx = [k0 + WIDE * ig + x for x in range(WIDE)]
            nx = [kx[x] + WIDE for x in range(WIDE)]
            gx = [kx[x] < k1 for x in range(WIDE)]
            hx = [nx[x] < k1 for x in range(WIDE)]

            for x in range(WIDE):
                @pl.when(gx[x])
                def _(x=x):
                    pltpu.make_async_copy(
                        itab_h.at[bi[x]], rows[x], semG[x]).wait()
                    process(lo, x)
                    pltpu.async_copy(rows[x], uacc_s.at[uofs[x]], semS[x],
                                     add=True)

            for x in range(WIDE):
                @pl.when(hx[x])
                def _(x=x):
                    fetch_idx(nx[x] * T, x)

            for x in range(WIDE):
                @pl.when(hx[x])
                def _(x=x):
                    wait_idx(nx[x] * T, x)
                    pltpu.make_async_copy(
                        rows[x], uacc_s.at[uofs[x]], semS[x]).wait()
                    pltpu.async_copy(itab_h.at[bi[x]], rows[x], semG[x])

            for x in range(WIDE):
                @pl.when(gx[x] & jnp.logical_not(hx[x]))
                def _(x=x):
                    pltpu.make_async_copy(
                        rows[x], uacc_s.at[uofs[x]], semS[x]).wait()

        def fetch_gs(q, y):
            ub = lo + q * C
            pltpu.async_copy(gu_h.at[pl.ds(ub, C)], gu[y], semB[y])
            pltpu.async_copy(seg_h.at[pl.ds(ub, C)], seg[y], semB[y])

        def wait_gs(q, y):
            ub = lo + q * C
            pltpu.make_async_copy(gu_h.at[pl.ds(ub, C)], gu[y], semB[y]).wait()
            pltpu.make_async_copy(seg_h.at[pl.ds(ub, C)], seg[y],
                                  semB[y]).wait()

        def process_b(q, y):
            guy, urowsy, accy, fmy = gu[y], urows[y], acc[y], fm[y]
            pltpu.sync_copy(uacc_s.at[pl.ds(sbase + q * C, C)], accy)
            for v8 in range(C // 16):
                sl = pl.ds(v8 * 16, 16)
                fmy[sl] = jnp.where(guy[sl] != 0, 1.0, 0.0)

            @pl.loop(0, C // 16)
            def _mul(b):
                base = b * 16
                fv16 = fmy[pl.ds(base, 16)]
                for rsub in range(16):
                    fv = jnp.broadcast_to(fv16[rsub], (16,))
                    rr = base + rsub
                    for h in (0, 16):
                        urowsy[rr, pl.ds(h, 16)] = (
                            urowsy[rr, pl.ds(h, 16)]
                            * accy[rr, pl.ds(h, 16)] * fv)

        @pl.loop(0, (NCHUNK + 1) // 2)
        def _bpair(i2):
            qa = 2 * i2
            qb = qa + 1
            gb = qb < NCHUNK

            fetch_gs(qa, 0)

            @pl.when(gb)
            def _():
                fetch_gs(qb, 1)

            wait_gs(qa, 0)
            pltpu.async_copy(utab_h.at[gu[0]], urows[0], semU[0])

            @pl.when(gb)
            def _():
                wait_gs(qb, 1)
                pltpu.async_copy(utab_h.at[gu[1]], urows[1], semU[1])

            pltpu.make_async_copy(utab_h.at[gu[0]], urows[0], semU[0]).wait()
            process_b(qa, 0)
            pltpu.async_copy(urows[0], gacc_s.at[seg[0]], semW[0], add=True)
            pltpu.async_copy(ones_v, gcnt_s.at[seg[0]], semW[0], add=True)

            @pl.when(gb)
            def _():
                pltpu.make_async_copy(
                    utab_h.at[gu[1]], urows[1], semU[1]).wait()
                process_b(qb, 1)
                pltpu.async_copy(urows[1], gacc_s.at[seg[1]], semW[1],
                                 add=True)
                pltpu.async_copy(ones_v, gcnt_s.at[seg[1]], semW[1], add=True)

            pltpu.make_async_copy(urows[0], gacc_s.at[seg[0]], semW[0]).wait()
            pltpu.make_async_copy(ones_v, gcnt_s.at[seg[0]], semW[0]).wait()

            @pl.when(gb)
            def _():
                pltpu.make_async_copy(
                    urows[1], gacc_s.at[seg[1]], semW[1]).wait()
                pltpu.make_async_copy(
                    ones_v, gcnt_s.at[seg[1]], semW[1]).wait()

    plsc.subcore_barrier()
    for t in range(GSLICE // T):
        off = s * GSLICE + t * T
        pltpu.sync_copy(gacc_s.at[pl.ds(off, T)], outp_h.at[c].at[pl.ds(off, T)])
        pltpu.sync_copy(gcnt_s.at[pl.ds(off, T)], outc_h.at[c].at[pl.ds(off, T)])


def _combine_body(p_ref, c_ref, o_ref):
    ps = p_ref[0] + p_ref[1]
    cnt = c_ref[0, :, 0:1] + c_ref[1, :, 0:1]
    o_ref[...] = ps / jnp.maximum(cnt, 1.0)


@jax.jit
def kernel(group_user_flat, user_seg_ids, behavior_items, behavior_counts,
           behavior_user_ids, user_table, item_table, lin_W, lin_b):
    del lin_W, lin_b
    bounds = jnp.arange(0, TOTAL_USERS + 1, UR, dtype=_i32)
    starts = jnp.searchsorted(behavior_user_ids, bounds).astype(_i32)
    starts = jnp.concatenate(
        [starts, jnp.full((SPAD - NRANGE - 1,), TOTAL_BEHAVIORS, _i32)])

    mesh = plsc.VectorSubcoreMesh(core_axis_name="c", subcore_axis_name="s")
    scratch = [
        pltpu.MemorySpace.VMEM_SHARED((NS * UR, EMB), _f32),
        pltpu.MemorySpace.VMEM_SHARED((N_GROUPS, EMB), _f32),
        pltpu.MemorySpace.VMEM_SHARED((N_GROUPS, 16), _f32),
        pltpu.VMEM((SPAD,), _i32),
    ]
    scratch += [pltpu.VMEM((T,), _i32) for _ in range(WIDE)]
    scratch += [pltpu.VMEM((T,), _f32) for _ in range(WIDE)]
    scratch += [pltpu.VMEM((T,), _i32) for _ in range(WIDE)]
    scratch += [pltpu.VMEM((T,), _i32) for _ in range(WIDE)]
    scratch += [pltpu.VMEM((T, EMB), _f32) for _ in range(WIDE)]
    scratch += [pltpu.VMEM((C,), _i32) for _ in range(2)]
    scratch += [pltpu.VMEM((C,), _i32) for _ in range(2)]
    scratch += [pltpu.VMEM((C, EMB), _f32) for _ in range(2)]
    scratch += [pltpu.VMEM((C, EMB), _f32) for _ in range(2)]
    scratch += [pltpu.VMEM((C,), _f32) for _ in range(2)]
    scratch += [
        pltpu.VMEM((T, EMB), _f32),
        pltpu.VMEM((T, 16), _f32),
        pltpu.VMEM((C, 16), _f32),
    ]
    scratch += [pltpu.SemaphoreType.DMA] * (3 * WIDE + 6)

    sc = pl.kernel(
        _sc_body,
        out_type=(
            jax.ShapeDtypeStruct((NC, N_GROUPS, EMB), _f32),
            jax.ShapeDtypeStruct((NC, N_GROUPS, 16), _f32),
        ),
        mesh=mesh,
        compiler_params=pltpu.CompilerParams(
            needs_layout_passes=False, use_tc_tiling_on_sc=False),
        scratch_types=scratch,
    )
    partials, cnts = sc(starts, group_user_flat, user_seg_ids, behavior_items,
                        behavior_counts, behavior_user_ids, user_table,
                        item_table)

    out = pl.pallas_call(
        _combine_body,
        out_shape=jax.ShapeDtypeStruct((N_GROUPS, EMB), _f32),
    )(partials, cnts)
    return out
